# baseline (speedup 1.0000x reference)
import functools

import jax
import jax.numpy as jnp
from jax.experimental import pallas as pl
from jax.experimental.pallas import tpu as pltpu

P = 8
N = 16384
D = 32
H = 32
A = 8
Q = N // P
BQ = 1024
PD = P * D
PH = P * H
G = 4 * PH


def _dot_t(a, bt):
    return jax.lax.dot_general(a, bt, (((1,), (1,)), ((), ())),
                               preferred_element_type=jnp.float32)


def _pool_body(x_ref, h_ref, c_ref, wih_ref, whh_ref, b_ref, wav_ref, bav_ref,
               act_ref, lgp_ref, ent_ref, val_ref, ho_ref, co_ref,
               wcat_s, bbig_s):
    @pl.when(pl.program_id(0) == 0)
    def _build():
        wcat_s[...] = jnp.zeros((G, PD + PH), jnp.float32)
        for p in range(P):
            for g in range(4):
                r = g * PH + H * p
                wcat_s[r:r + H, D * p:D * (p + 1)] = \
                    wih_ref[p, g * H:(g + 1) * H, :]
                wcat_s[r:r + H, PD + H * p:PD + H * (p + 1)] = \
                    whh_ref[p, g * H:(g + 1) * H, :]
                bbig_s[0:1, r:r + H] = b_ref[p:p + 1, g * H:(g + 1) * H]

    x = x_ref[...].reshape(BQ, PD)
    h = h_ref[...].reshape(BQ, PH)
    c = c_ref[...].reshape(BQ, PH)
    xh = jnp.concatenate([x, h], axis=1)
    gates = _dot_t(xh, wcat_s[...]) + bbig_s[...]
    sif = jax.nn.sigmoid(gates[:, 0:2 * PH])
    ig = sif[:, 0:PH]
    fg = sif[:, PH:2 * PH]
    gg = jnp.tanh(gates[:, 2 * PH:3 * PH])
    og = jax.nn.sigmoid(gates[:, 3 * PH:4 * PH])
    c_new = fg * c + ig * gg
    h_new = og * jnp.tanh(c_new)
    ho_ref[...] = h_new.reshape(BQ, P, H)
    co_ref[...] = c_new.reshape(BQ, P, H)

    acts, lgps, ents, vals = [], [], [], []
    fiota = jax.lax.broadcasted_iota(jnp.int32, (BQ, A), 1).astype(jnp.float32)
    for p in range(P):
        hp = h_new[:, H * p:H * (p + 1)]
        la = _dot_t(hp, wav_ref[p]) + bav_ref[p:p + 1, :]
        logits = la[:, :A]
        m = jnp.max(logits, axis=-1, keepdims=True)
        d = logits - m
        ex = jnp.exp(d)
        s = jnp.sum(ex, axis=-1, keepdims=True)
        t = jnp.sum(ex * d, axis=-1, keepdims=True)
        logs = jnp.log(s)
        atn_f = jnp.min(jnp.where(d == 0.0, fiota, float(A)),
                        axis=-1, keepdims=True)
        acts.append(atn_f.astype(jnp.int32))
        lgps.append(-logs)
        ents.append(logs - t / s)
        vals.append(la[:, A:A + 1])
    act_ref[...] = jnp.concatenate(acts, axis=1)
    lgp_ref[...] = jnp.concatenate(lgps, axis=1)
    ent_ref[...] = jnp.concatenate(ents, axis=1)
    val_ref[...] = jnp.concatenate(vals, axis=1)


@functools.partial(jax.jit, static_argnames=("interpret",))
def _run(obs, lstm_h, lstm_c, W_ih, W_hh, b, W_a, b_a, W_v, b_v,
         interpret=False):
    x3 = obs.reshape(Q, P, D)
    h3 = lstm_h.reshape(Q, P, H)
    c3 = lstm_c.reshape(Q, P, H)
    wav = jnp.concatenate([W_a, W_v], axis=1)
    bav = jnp.concatenate([b_a, b_v], axis=1)

    grid = (Q // BQ,)
    row3 = lambda d2: pl.BlockSpec((BQ, P, d2), lambda i: (i, 0, 0))
    row2 = lambda d2: pl.BlockSpec((BQ, d2), lambda i: (i, 0))
    full = lambda shape: pl.BlockSpec(shape, lambda i: (0,) * len(shape))

    out_shapes = (
        jax.ShapeDtypeStruct((Q, P), jnp.int32),
        jax.ShapeDtypeStruct((Q, P), jnp.float32),
        jax.ShapeDtypeStruct((Q, P), jnp.float32),
        jax.ShapeDtypeStruct((Q, P), jnp.float32),
        jax.ShapeDtypeStruct((Q, P, H), jnp.float32),
        jax.ShapeDtypeStruct((Q, P, H), jnp.float32),
    )
    act, lgp, ent, val, ho, co = pl.pallas_call(
        _pool_body,
        grid=grid,
        in_specs=[
            row3(D), row3(H), row3(H),
            full((P, 4 * H, D)), full((P, 4 * H, H)), full((P, 4 * H)),
            full((P, A + 1, H)), full((P, A + 1)),
        ],
        out_specs=[
            row2(P), row2(P), row2(P), row2(P), row3(H), row3(H),
        ],
        out_shape=out_shapes,
        scratch_shapes=[
            pltpu.VMEM((G, PD + PH), jnp.float32),
            pltpu.VMEM((1, G), jnp.float32),
        ],
        interpret=interpret,
    )(x3, h3, c3, W_ih, W_hh, b, wav, bav)

    return (act.reshape(N), lgp.reshape(N), ent.reshape(N), val.reshape(N),
            (ho.reshape(1, N, H), co.reshape(1, N, H)))


def kernel(obs, lstm_h, lstm_c, W_ih, W_hh, b, W_a, b_a, W_v, b_v):
    return _run(obs, lstm_h, lstm_c, W_ih, W_hh, b, W_a, b_a, W_v, b_v)


# speedup vs baseline: 1.1365x; 1.1365x over previous
import functools

import jax
import jax.numpy as jnp
from jax.experimental import pallas as pl
from jax.experimental.pallas import tpu as pltpu

P = 8
N = 16384
D = 32
H = 32
A = 8
Q = N // P
BQ = 512
PD = P * D
PH = P * H
G = 4 * PH


def _dot_t(a, bt):
    return jax.lax.dot_general(a, bt, (((1,), (1,)), ((), ())),
                               preferred_element_type=jnp.float32)


def _pool_body(x_ref, h_ref, c_ref, wih_ref, whh_ref, b_ref, wav_ref, bav_ref,
               act_ref, lgp_ref, ent_ref, val_ref, ho_ref, co_ref,
               wcat_s, bbig_s, whd_s, bhd_s, esel_s):
    @pl.when(pl.program_id(0) == 0)
    def _build():
        wcat_s[...] = jnp.zeros((G, PD + PH), jnp.float32)
        whd_s[...] = jnp.zeros((2 * A * P, PH), jnp.float32)
        bhd_s[...] = jnp.zeros((1, 2 * A * P), jnp.float32)
        esel_s[...] = jnp.zeros((2 * A * P, 2 * A), jnp.float32)
        one = jnp.ones((1, 1), jnp.float32)
        for p in range(P):
            whd_s[2 * A * p:2 * A * p + A + 1, H * p:H * (p + 1)] = \
                wav_ref[p]
            bhd_s[0:1, 2 * A * p:2 * A * p + A + 1] = bav_ref[p:p + 1, :]
            esel_s[2 * A * p:2 * A * p + 1, p:p + 1] = one
            esel_s[2 * A * p + A:2 * A * p + A + 1, A + p:A + p + 1] = one
            for g in range(4):
                r = g * PH + H * p
                wcat_s[r:r + H, D * p:D * (p + 1)] = \
                    wih_ref[p, g * H:(g + 1) * H, :]
                wcat_s[r:r + H, PD + H * p:PD + H * (p + 1)] = \
                    whh_ref[p, g * H:(g + 1) * H, :]
                bbig_s[0:1, r:r + H] = b_ref[p:p + 1, g * H:(g + 1) * H]

    x = x_ref[...].reshape(BQ, PD)
    h = h_ref[...].reshape(BQ, PH)
    c = c_ref[...].reshape(BQ, PH)
    xh = jnp.concatenate([x, h], axis=1)
    gates = _dot_t(xh, wcat_s[...]) + bbig_s[...]
    sif = jax.nn.sigmoid(gates[:, 0:2 * PH])
    ig = sif[:, 0:PH]
    fg = sif[:, PH:2 * PH]
    gg = jnp.tanh(gates[:, 2 * PH:3 * PH])
    og = jax.nn.sigmoid(gates[:, 3 * PH:4 * PH])
    c_new = fg * c + ig * gg
    h_new = og * jnp.tanh(c_new)
    ho_ref[...] = h_new.reshape(BQ, P, H)
    co_ref[...] = c_new.reshape(BQ, P, H)

    la = _dot_t(h_new, whd_s[...]) + bhd_s[...]
    W = 2 * A * P
    lane = jax.lax.broadcasted_iota(jnp.int32, (BQ, W), 1)
    k16 = lane % (2 * A)
    lm = k16 < A
    v0 = jnp.where(lm, la, -1e30)
    s = jnp.maximum(v0, jnp.roll(v0, -1, axis=1))
    s = jnp.maximum(s, jnp.roll(s, -2, axis=1))
    s = jnp.maximum(s, jnp.roll(s, -4, axis=1))
    mb = jnp.maximum(s, jnp.roll(s, 1, axis=1))
    mb = jnp.maximum(mb, jnp.roll(mb, 2, axis=1))
    mb = jnp.maximum(mb, jnp.roll(mb, 4, axis=1))
    d = la - mb
    exv = jnp.where(lm, jnp.exp(d), 0.0)
    ss = exv + jnp.roll(exv, -1, axis=1)
    ss = ss + jnp.roll(ss, -2, axis=1)
    ss = ss + jnp.roll(ss, -4, axis=1)
    tv = exv * (mb - la)
    ts = tv + jnp.roll(tv, -1, axis=1)
    ts = ts + jnp.roll(ts, -2, axis=1)
    ts = ts + jnp.roll(ts, -4, axis=1)
    ind = jnp.where(lm & (d == 0.0), k16.astype(jnp.float32), float(A))
    im = jnp.minimum(ind, jnp.roll(ind, -1, axis=1))
    im = jnp.minimum(im, jnp.roll(im, -2, axis=1))
    im = jnp.minimum(im, jnp.roll(im, -4, axis=1))
    sel0 = esel_s[:, 0:A]
    selv = esel_s[:, A:2 * A]
    dd0 = (((1,), (0,)), ((), ()))
    s8 = jax.lax.dot_general(ss, sel0, dd0, preferred_element_type=jnp.float32)
    t8 = jax.lax.dot_general(ts, sel0, dd0, preferred_element_type=jnp.float32)
    a8 = jax.lax.dot_general(im, sel0, dd0, preferred_element_type=jnp.float32)
    v8 = jax.lax.dot_general(la, selv, dd0, preferred_element_type=jnp.float32)
    logs = jnp.log(s8)
    act_ref[...] = a8.astype(jnp.int32)
    lgp_ref[...] = -logs
    ent_ref[...] = logs + t8 / s8
    val_ref[...] = v8


@functools.partial(jax.jit, static_argnames=("interpret",))
def _run(obs, lstm_h, lstm_c, W_ih, W_hh, b, W_a, b_a, W_v, b_v,
         interpret=False):
    x3 = obs.reshape(Q, P, D)
    h3 = lstm_h.reshape(Q, P, H)
    c3 = lstm_c.reshape(Q, P, H)
    wav = jnp.concatenate([W_a, W_v], axis=1)
    bav = jnp.concatenate([b_a, b_v], axis=1)

    grid = (Q // BQ,)
    row3 = lambda d2: pl.BlockSpec((BQ, P, d2), lambda i: (i, 0, 0))
    row2 = lambda d2: pl.BlockSpec((BQ, d2), lambda i: (i, 0))
    full = lambda shape: pl.BlockSpec(shape, lambda i: (0,) * len(shape))

    out_shapes = (
        jax.ShapeDtypeStruct((Q, P), jnp.int32),
        jax.ShapeDtypeStruct((Q, P), jnp.float32),
        jax.ShapeDtypeStruct((Q, P), jnp.float32),
        jax.ShapeDtypeStruct((Q, P), jnp.float32),
        jax.ShapeDtypeStruct((Q, P, H), jnp.float32),
        jax.ShapeDtypeStruct((Q, P, H), jnp.float32),
    )
    act, lgp, ent, val, ho, co = pl.pallas_call(
        _pool_body,
        grid=grid,
        in_specs=[
            row3(D), row3(H), row3(H),
            full((P, 4 * H, D)), full((P, 4 * H, H)), full((P, 4 * H)),
            full((P, A + 1, H)), full((P, A + 1)),
        ],
        out_specs=[
            row2(P), row2(P), row2(P), row2(P), row3(H), row3(H),
        ],
        out_shape=out_shapes,
        scratch_shapes=[
            pltpu.VMEM((G, PD + PH), jnp.float32),
            pltpu.VMEM((1, G), jnp.float32),
            pltpu.VMEM((2 * A * P, PH), jnp.float32),
            pltpu.VMEM((1, 2 * A * P), jnp.float32),
            pltpu.VMEM((2 * A * P, 2 * A), jnp.float32),
        ],
        interpret=interpret,
    )(x3, h3, c3, W_ih, W_hh, b, wav, bav)

    return (act.reshape(N), lgp.reshape(N), ent.reshape(N), val.reshape(N),
            (ho.reshape(1, N, H), co.reshape(1, N, H)))


def kernel(obs, lstm_h, lstm_c, W_ih, W_hh, b, W_a, b_a, W_v, b_v):
    return _run(obs, lstm_h, lstm_c, W_ih, W_hh, b, W_a, b_a, W_v, b_v)
